# Initial kernel scaffold; baseline (speedup 1.0000x reference)
import jax
import jax.numpy as jnp
from jax.experimental import pallas as pl


def kernel(ew1, eb1, ew2, eb2, pw, pb, dw1, db1, dw2, db2, x, edge_index, edge_weight):
    raise NotImplementedError("write your pallas kernel here")


# speedup vs baseline: 1.4149x; 1.4149x over previous
import functools

import jax
import jax.numpy as jnp
from jax.experimental import pallas as pl
from jax.experimental.pallas import tpu as pltpu

_INPUT = 2
_HID = 64
_VMEM_LIMIT = 56 * 1024 * 1024
_ROW_BLK = 512


def _encoder_body(x_ref, w1_ref, b1_ref, w2_ref, b2_ref, h_ref):
    x = x_ref[...]
    h1 = x[:, 0:1] * w1_ref[0:1, :] + x[:, 1:2] * w1_ref[1:2, :] + b1_ref[...]
    h1 = jnp.maximum(h1, 0.0)
    h2 = jnp.dot(h1.astype(jnp.bfloat16), w2_ref[...],
                 preferred_element_type=jnp.float32) + b2_ref[...]
    h_ref[...] = h2.astype(h_ref.dtype)


def _gcn_body(a_ref, h_ref, w_ref, b_ref, out_ref):
    agg = jnp.dot(a_ref[...], h_ref[...], preferred_element_type=jnp.float32)
    out = jnp.dot(agg.astype(jnp.bfloat16), w_ref[...],
                  preferred_element_type=jnp.float32) + b_ref[...]
    out_ref[...] = out.astype(out_ref.dtype)


def _gcn_decoder_body(a_ref, h_ref, w_ref, b_ref, dw1_ref, db1_ref,
                      dw2_ref, db2_ref, out_ref):
    agg = jnp.dot(a_ref[...], h_ref[...], preferred_element_type=jnp.float32)
    h3 = jnp.dot(agg.astype(jnp.bfloat16), w_ref[...],
                 preferred_element_type=jnp.float32) + b_ref[...]
    d = jnp.dot(h3.astype(jnp.bfloat16), dw1_ref[...],
                preferred_element_type=jnp.float32) + db1_ref[...]
    d = jnp.maximum(d, 0.0)
    o = jnp.sum(d * dw2_ref[...], axis=-1, keepdims=True) + db2_ref[...]
    out_ref[...] = jax.nn.sigmoid(o)


def _dense_norm_adj(edge_index, edge_weight, num_nodes, n_pad):
    row = edge_index[0]
    col = edge_index[1]
    loop = jnp.arange(num_nodes, dtype=edge_index.dtype)
    row = jnp.concatenate([row, loop])
    col = jnp.concatenate([col, loop])
    w = jnp.concatenate([edge_weight, jnp.ones((num_nodes,), edge_weight.dtype)])
    deg = jnp.zeros((num_nodes,), w.dtype).at[col].add(w)
    dinv = jnp.where(deg > 0, 1.0 / jnp.sqrt(deg), 0.0)
    norm = dinv[row] * w * dinv[col]
    a = jnp.zeros((n_pad, n_pad), w.dtype).at[col, row].add(norm)
    return a


@functools.partial(jax.jit, static_argnames=())
def _forward(ew1, eb1, ew2, eb2, pw, pb, dw1, db1, dw2, db2,
             x, edge_index, edge_weight):
    n = x.shape[0]
    n_pad = ((n + _ROW_BLK - 1) // _ROW_BLK) * _ROW_BLK

    a = _dense_norm_adj(edge_index, edge_weight, n, n_pad).astype(jnp.bfloat16)
    x_pad = jnp.zeros((n_pad, _INPUT), jnp.float32).at[:n].set(x)

    enc_tile = min(n_pad, 4096)
    h = pl.pallas_call(
        _encoder_body,
        out_shape=jax.ShapeDtypeStruct((n_pad, _HID), jnp.bfloat16),
        grid=(n_pad // enc_tile,),
        in_specs=[
            pl.BlockSpec((enc_tile, _INPUT), lambda i: (i, 0)),
            pl.BlockSpec((_INPUT, _HID), lambda i: (0, 0)),
            pl.BlockSpec((1, _HID), lambda i: (0, 0)),
            pl.BlockSpec((_HID, _HID), lambda i: (0, 0)),
            pl.BlockSpec((1, _HID), lambda i: (0, 0)),
        ],
        out_specs=pl.BlockSpec((enc_tile, _HID), lambda i: (i, 0)),
        compiler_params=pltpu.CompilerParams(
            dimension_semantics=("parallel",),
            vmem_limit_bytes=_VMEM_LIMIT),
    )(x_pad, ew1, eb1, ew2.astype(jnp.bfloat16), eb2)

    grid = (n_pad // _ROW_BLK,)
    gcn_specs = [
        pl.BlockSpec((_ROW_BLK, n_pad), lambda i: (i, 0)),
        pl.BlockSpec((n_pad, _HID), lambda i: (0, 0)),
        pl.BlockSpec((_HID, _HID), lambda i: (0, 0)),
        pl.BlockSpec((1, _HID), lambda i: (0, 0)),
    ]
    for l in range(2):
        h = pl.pallas_call(
            _gcn_body,
            out_shape=jax.ShapeDtypeStruct((n_pad, _HID), jnp.bfloat16),
            grid=grid,
            in_specs=gcn_specs,
            out_specs=pl.BlockSpec((_ROW_BLK, _HID), lambda i: (i, 0)),
            compiler_params=pltpu.CompilerParams(
                dimension_semantics=("parallel",),
                vmem_limit_bytes=_VMEM_LIMIT),
        )(a, h, pw[l].astype(jnp.bfloat16), pb[l])

    out = pl.pallas_call(
        _gcn_decoder_body,
        out_shape=jax.ShapeDtypeStruct((n_pad, 1), jnp.float32),
        grid=grid,
        in_specs=gcn_specs + [
            pl.BlockSpec((_HID, _HID), lambda i: (0, 0)),
            pl.BlockSpec((1, _HID), lambda i: (0, 0)),
            pl.BlockSpec((1, _HID), lambda i: (0, 0)),
            pl.BlockSpec((1, 1), lambda i: (0, 0)),
        ],
        out_specs=pl.BlockSpec((_ROW_BLK, 1), lambda i: (i, 0)),
        compiler_params=pltpu.CompilerParams(
            dimension_semantics=("parallel",),
            vmem_limit_bytes=_VMEM_LIMIT),
    )(a, h, pw[2].astype(jnp.bfloat16), pb[2],
      dw1.astype(jnp.bfloat16), db1, dw2.T, db2)

    return out[:n]


def kernel(ew1, eb1, ew2, eb2, pw, pb, dw1, db1, dw2, db2,
           x, edge_index, edge_weight):
    return _forward(ew1, eb1, ew2, eb2, pw, pb, dw1, db1, dw2, db2,
                    x, edge_index, edge_weight)


# speedup vs baseline: 6.2533x; 4.4197x over previous
import jax
import jax.numpy as jnp
from jax.experimental import pallas as pl
from jax.experimental.pallas import tpu as pltpu

_INPUT = 2
_HID = 64
_VMEM_LIMIT = 56 * 1024 * 1024
_ROW_BLK = 512
_NORM_BLK = 128


def _normalize_body(a_ref, dinv_blk_ref, dinv_row_ref, out_ref):
    anorm = a_ref[...] * dinv_blk_ref[...] * dinv_row_ref[...]
    out_ref[...] = anorm.astype(out_ref.dtype)


def _encoder_body(x_ref, w1_ref, b1_ref, w2_ref, b2_ref, h_ref):
    x = x_ref[...]
    h1 = x[:, 0:1] * w1_ref[0:1, :] + x[:, 1:2] * w1_ref[1:2, :] + b1_ref[...]
    h1 = jnp.maximum(h1, 0.0)
    h2 = jnp.dot(h1.astype(jnp.bfloat16), w2_ref[...],
                 preferred_element_type=jnp.float32) + b2_ref[...]
    h_ref[...] = h2.astype(h_ref.dtype)


def _gcn_body(a_ref, h_ref, hblk_ref, dinv_ref, w_ref, b_ref, out_ref):
    agg = jnp.dot(a_ref[...], h_ref[...], preferred_element_type=jnp.float32)
    agg += (dinv_ref[...] * dinv_ref[...]) * hblk_ref[...].astype(jnp.float32)
    out = jnp.dot(agg.astype(jnp.bfloat16), w_ref[...],
                  preferred_element_type=jnp.float32) + b_ref[...]
    out_ref[...] = out.astype(out_ref.dtype)


def _gcn_decoder_body(a_ref, h_ref, hblk_ref, dinv_ref, w_ref, b_ref,
                      dw1_ref, db1_ref, dw2_ref, db2_ref, out_ref):
    agg = jnp.dot(a_ref[...], h_ref[...], preferred_element_type=jnp.float32)
    agg += (dinv_ref[...] * dinv_ref[...]) * hblk_ref[...].astype(jnp.float32)
    h3 = jnp.dot(agg.astype(jnp.bfloat16), w_ref[...],
                 preferred_element_type=jnp.float32) + b_ref[...]
    d = jnp.dot(h3.astype(jnp.bfloat16), dw1_ref[...],
                preferred_element_type=jnp.float32) + db1_ref[...]
    d = jnp.maximum(d, 0.0)
    o = jnp.sum(d * dw2_ref[...], axis=-1, keepdims=True) + db2_ref[...]
    out_ref[...] = jax.nn.sigmoid(o)


@jax.jit
def _forward(ew1, eb1, ew2, eb2, pw, pb, dw1, db1, dw2, db2,
             x, edge_index, edge_weight):
    n = x.shape[0]
    n_pad = ((n + _ROW_BLK - 1) // _ROW_BLK) * _ROW_BLK

    src = edge_index[0]
    dst = edge_index[1]
    a_raw = jnp.zeros((n_pad, n_pad), jnp.float32).at[dst, src].add(edge_weight)
    deg = jnp.zeros((n,), jnp.float32).at[dst].add(edge_weight) + 1.0
    dinv = jax.lax.rsqrt(deg)
    dinv_col = jnp.zeros((n_pad, 1), jnp.float32).at[:n, 0].set(dinv)
    dinv_row = jnp.zeros((1, n_pad), jnp.float32).at[0, :n].set(dinv)
    x_pad = jnp.zeros((n_pad, _INPUT), jnp.float32).at[:n].set(x)

    a = pl.pallas_call(
        _normalize_body,
        out_shape=jax.ShapeDtypeStruct((n_pad, n_pad), jnp.bfloat16),
        grid=(n_pad // _NORM_BLK,),
        in_specs=[
            pl.BlockSpec((_NORM_BLK, n_pad), lambda i: (i, 0)),
            pl.BlockSpec((_NORM_BLK, 1), lambda i: (i, 0)),
            pl.BlockSpec((1, n_pad), lambda i: (0, 0)),
        ],
        out_specs=pl.BlockSpec((_NORM_BLK, n_pad), lambda i: (i, 0)),
        compiler_params=pltpu.CompilerParams(
            dimension_semantics=("parallel",),
            vmem_limit_bytes=_VMEM_LIMIT),
    )(a_raw, dinv_col, dinv_row)

    enc_tile = min(n_pad, 4096)
    h = pl.pallas_call(
        _encoder_body,
        out_shape=jax.ShapeDtypeStruct((n_pad, _HID), jnp.bfloat16),
        grid=(n_pad // enc_tile,),
        in_specs=[
            pl.BlockSpec((enc_tile, _INPUT), lambda i: (i, 0)),
            pl.BlockSpec((_INPUT, _HID), lambda i: (0, 0)),
            pl.BlockSpec((1, _HID), lambda i: (0, 0)),
            pl.BlockSpec((_HID, _HID), lambda i: (0, 0)),
            pl.BlockSpec((1, _HID), lambda i: (0, 0)),
        ],
        out_specs=pl.BlockSpec((enc_tile, _HID), lambda i: (i, 0)),
        compiler_params=pltpu.CompilerParams(
            dimension_semantics=("parallel",),
            vmem_limit_bytes=_VMEM_LIMIT),
    )(x_pad, ew1, eb1, ew2.astype(jnp.bfloat16), eb2)

    grid = (n_pad // _ROW_BLK,)
    gcn_specs = [
        pl.BlockSpec((_ROW_BLK, n_pad), lambda i: (i, 0)),
        pl.BlockSpec((n_pad, _HID), lambda i: (0, 0)),
        pl.BlockSpec((_ROW_BLK, _HID), lambda i: (i, 0)),
        pl.BlockSpec((_ROW_BLK, 1), lambda i: (i, 0)),
        pl.BlockSpec((_HID, _HID), lambda i: (0, 0)),
        pl.BlockSpec((1, _HID), lambda i: (0, 0)),
    ]
    for l in range(2):
        h = pl.pallas_call(
            _gcn_body,
            out_shape=jax.ShapeDtypeStruct((n_pad, _HID), jnp.bfloat16),
            grid=grid,
            in_specs=gcn_specs,
            out_specs=pl.BlockSpec((_ROW_BLK, _HID), lambda i: (i, 0)),
            compiler_params=pltpu.CompilerParams(
                dimension_semantics=("parallel",),
                vmem_limit_bytes=_VMEM_LIMIT),
        )(a, h, h, dinv_col, pw[l].astype(jnp.bfloat16), pb[l])

    out = pl.pallas_call(
        _gcn_decoder_body,
        out_shape=jax.ShapeDtypeStruct((n_pad, 1), jnp.float32),
        grid=grid,
        in_specs=gcn_specs + [
            pl.BlockSpec((_HID, _HID), lambda i: (0, 0)),
            pl.BlockSpec((1, _HID), lambda i: (0, 0)),
            pl.BlockSpec((1, _HID), lambda i: (0, 0)),
            pl.BlockSpec((1, 1), lambda i: (0, 0)),
        ],
        out_specs=pl.BlockSpec((_ROW_BLK, 1), lambda i: (i, 0)),
        compiler_params=pltpu.CompilerParams(
            dimension_semantics=("parallel",),
            vmem_limit_bytes=_VMEM_LIMIT),
    )(a, h, h, dinv_col, pw[2].astype(jnp.bfloat16), pb[2],
      dw1.astype(jnp.bfloat16), db1, dw2.T, db2)

    return out[:n]


def kernel(ew1, eb1, ew2, eb2, pw, pb, dw1, db1, dw2, db2,
           x, edge_index, edge_weight):
    return _forward(ew1, eb1, ew2, eb2, pw, pb, dw1, db1, dw2, db2,
                    x, edge_index, edge_weight)
